# baseline (speedup 1.0000x reference)
import functools

import jax
import jax.numpy as jnp
from jax import lax
from jax.experimental import pallas as pl
from jax.experimental.pallas import tpu as pltpu
from jax.experimental.pallas import tpu_sc as plsc

_B = 1024
_D = 100
_DEN = 1003
_NPAIR = 4 * _D
_NW = 32
_RPW = _B // _NW
_L = 16
_G = 5
_CP = _NPAIR // _G


def _sc_body(x_hbm, dtab_hbm, etab_hbm, out_hbm,
             xv, uv, sv, idxv, wv, dbufs, ebufs, tv, ev, outv,
             xsem, gsems):
    wid = lax.axis_index("s") * 2 + lax.axis_index("c")
    lanes = lax.iota(jnp.int32, _L)
    row0 = wid * _RPW

    def load_x(i, par):
        return pltpu.async_copy(
            x_hbm.at[pl.ds((row0 + i) * 104, 104)],
            xv[par].at[pl.ds(0, 104)], xsem)

    def wait_x(par):
        pltpu.make_async_copy(
            x_hbm.at[pl.ds(0, 104)], xv[par].at[pl.ds(0, 104)], xsem).wait()

    def compute_idx_w(par):
        for c in range(7):
            xx = xv[par][pl.ds(_L * c, _L)]
            t = xx * 1000.0
            bse = t.astype(jnp.int32)
            uv[pl.ds(_L * c, _L)] = t - bse.astype(jnp.float32)
            sv[pl.ds(_L * c, _L)] = (_L * c + lanes) * _DEN + bse
        for c in range(25):
            p = _L * c + lanes
            dq = lax.shift_right_logical(p, 2)
            j = lax.bitwise_and(p, 3)
            sg = plsc.load_gather(sv, [dq])
            ug = plsc.load_gather(uv, [dq])
            u2 = ug * ug
            u3 = u2 * ug
            om = 1.0 - ug
            w0 = om * om * om * (1.0 / 6.0)
            w1 = 0.5 * u3 - u2 + (2.0 / 3.0)
            w2 = -0.5 * u3 + 0.5 * u2 + 0.5 * ug + (1.0 / 6.0)
            w3 = u3 * (1.0 / 6.0)
            w = jnp.where(j == 0, w0,
                          jnp.where(j == 1, w1,
                                    jnp.where(j == 2, w2, w3)))
            wv[par][pl.ds(_L * c, _L)] = w
            idxv[par][(_L * c) // _CP, pl.ds((_L * c) % _CP, _L)] = sg + j

    def fire_chunk(par, g):
        pltpu.async_copy(dtab_hbm.at[idxv[par].at[g]], dbufs[g], gsems[g])
        pltpu.async_copy(etab_hbm.at[idxv[par].at[g]], ebufs[g], gsems[g])

    def wait_chunk(g):
        pltpu.make_async_copy(
            dtab_hbm.at[pl.ds(0, _CP)], dbufs[g], gsems[g]).wait()
        pltpu.make_async_copy(
            etab_hbm.at[pl.ds(0, _CP)], ebufs[g], gsems[g]).wait()

    def row_phase(i, cur, nxt):
        have_next = i + 1 < _RPW

        @pl.when(have_next)
        def _():
            wait_x(nxt)
            compute_idx_w(nxt)

        @pl.when(i + 2 < _RPW)
        def _():
            load_x(i + 2, cur)

        def mul2(wbc, v32):
            u = plsc.bitcast(v32, jnp.int32)
            lo = plsc.bitcast(jnp.left_shift(u, 16), jnp.float32)
            hi = plsc.bitcast(
                lax.bitwise_and(u, jnp.full((_L,), -65536, jnp.int32)),
                jnp.float32)
            return wbc * lo, wbc * hi

        zero = jnp.zeros((_L,), jnp.float32)
        acc = (zero,) * 10
        for g in range(_G):
            wait_chunk(g)

            def pair_body(q, carry, g=g):
                accs = list(carry)
                for k in range(4):
                    p = 4 * q + k
                    wbc = plsc.load_gather(
                        wv[cur],
                        [jnp.full((_L,), _CP * g + 0, jnp.int32) + p])
                    for m in range(5):
                        if m == 0:
                            v32 = dbufs[g][p, pl.ds(0, 2 * _L)]
                        else:
                            v32 = ebufs[g][p, pl.ds(2 * _L * (m - 1), 2 * _L)]
                        e, o = mul2(wbc, v32)
                        accs[2 * m] = accs[2 * m] + e
                        accs[2 * m + 1] = accs[2 * m + 1] + o
                return tuple(accs)

            acc = lax.fori_loop(0, _CP // 4, pair_body, acc)

            @pl.when(have_next)
            def _(g=g):
                fire_chunk(nxt, g)

        for m in range(5):
            plsc.store_scatter(tv, [32 * m + 2 * lanes], acc[2 * m])
            plsc.store_scatter(tv, [32 * m + 1 + 2 * lanes], acc[2 * m + 1])

        for m in range(8):
            ev[pl.ds(_L * m, _L)] = jnp.exp(tv[pl.ds(32 + _L * m, _L)])
        for h in range(2):
            o4 = (_L * h + lanes) * 4
            g0 = plsc.load_gather(ev, [o4])
            g1 = plsc.load_gather(ev, [o4 + 1])
            g2 = plsc.load_gather(ev, [o4 + 2])
            g3 = plsc.load_gather(ev, [o4 + 3])
            outv[pl.ds(32 * i + _L * h, _L)] = \
                (tv[pl.ds(_L * h, _L)] + g0 - g2) + 0.25 * (g1 - g3)

    xv[0][pl.ds(96, _L)] = jnp.full((_L,), 0.5, jnp.float32)
    xv[1][pl.ds(96, _L)] = jnp.full((_L,), 0.5, jnp.float32)

    load_x(0, 0)
    wait_x(0)
    compute_idx_w(0)
    load_x(1, 1)
    for g in range(_G):
        fire_chunk(0, g)

    def two_rows(ii, _):
        row_phase(2 * ii, 0, 1)
        row_phase(2 * ii + 1, 1, 0)
        return 0

    lax.fori_loop(0, _RPW // 2, two_rows, 0)

    pltpu.sync_copy(outv, out_hbm.at[pl.ds(row0 * 32, _RPW * 32)])


_sc_kernel = functools.partial(
    pl.kernel,
    mesh=plsc.VectorSubcoreMesh(core_axis_name="c", subcore_axis_name="s"),
    out_type=jax.ShapeDtypeStruct((_B * 32,), jnp.float32),
    compiler_params=pltpu.CompilerParams(
        needs_layout_passes=False, use_tc_tiling_on_sc=False),
    scratch_types=[
        [pltpu.VMEM((112,), jnp.float32) for _ in range(2)],
        pltpu.VMEM((112,), jnp.float32),
        pltpu.VMEM((112,), jnp.int32),
        [pltpu.VMEM((_G, _CP), jnp.int32) for _ in range(2)],
        [pltpu.VMEM((_NPAIR,), jnp.float32) for _ in range(2)],
        [pltpu.VMEM((_CP, 32), jnp.bfloat16) for _ in range(_G)],
        [pltpu.VMEM((_CP, 128), jnp.bfloat16) for _ in range(_G)],
        pltpu.VMEM((160,), jnp.float32),
        pltpu.VMEM((128,), jnp.float32),
        pltpu.VMEM((_RPW * 32,), jnp.float32),
        pltpu.SemaphoreType.DMA,
        [pltpu.SemaphoreType.DMA for _ in range(_G)],
    ],
)(_sc_body)


def kernel(input_tensor, direct_table, exp_table):
    xp = jnp.pad(input_tensor, ((0, 0), (0, 4))).reshape(-1)
    out = _sc_kernel(xp,
                     direct_table.astype(jnp.bfloat16),
                     exp_table.astype(jnp.bfloat16))
    return out.reshape(_B, 32)


# speedup vs baseline: 1.1620x; 1.1620x over previous
import functools

import jax
import jax.numpy as jnp
from jax import lax
from jax.experimental import pallas as pl
from jax.experimental.pallas import tpu as pltpu
from jax.experimental.pallas import tpu_sc as plsc

_B = 1024
_D = 100
_DEN = 1003
_NPAIR = 4 * _D
_NW = 32
_RPW = _B // _NW
_L = 16
_G = 5
_CP = _NPAIR // _G


def _sc_body(x_hbm, dtab_hbm, etab_hbm, out_hbm,
             xv, uv, sv, idxv, wv, dbufs, ebufs, tv, outv,
             xsem, gsems):
    wid = lax.axis_index("s") * 2 + lax.axis_index("c")
    lanes = lax.iota(jnp.int32, _L)
    row0 = wid * _RPW

    def load_x(i, par):
        return pltpu.async_copy(
            x_hbm.at[pl.ds((row0 + i) * 104, 104)],
            xv[par].at[pl.ds(0, 104)], xsem)

    def wait_x(par):
        pltpu.make_async_copy(
            x_hbm.at[pl.ds(0, 104)], xv[par].at[pl.ds(0, 104)], xsem).wait()

    def compute_idx_w(par):
        for c in range(7):
            xx = xv[par][pl.ds(_L * c, _L)]
            t = xx * 1000.0
            bse = t.astype(jnp.int32)
            uv[pl.ds(_L * c, _L)] = t - bse.astype(jnp.float32)
            sv[pl.ds(_L * c, _L)] = (_L * c + lanes) * _DEN + bse
        for c in range(25):
            p = _L * c + lanes
            dq = lax.shift_right_logical(p, 2)
            j = lax.bitwise_and(p, 3)
            sg = plsc.load_gather(sv, [dq])
            ug = plsc.load_gather(uv, [dq])
            u2 = ug * ug
            u3 = u2 * ug
            om = 1.0 - ug
            w0 = om * om * om * (1.0 / 6.0)
            w1 = 0.5 * u3 - u2 + (2.0 / 3.0)
            w2 = -0.5 * u3 + 0.5 * u2 + 0.5 * ug + (1.0 / 6.0)
            w3 = u3 * (1.0 / 6.0)
            w = jnp.where(j == 0, w0,
                          jnp.where(j == 1, w1,
                                    jnp.where(j == 2, w2, w3)))
            wv[par][pl.ds(_L * c, _L)] = w
            idxv[par][(_L * c) // _CP, pl.ds((_L * c) % _CP, _L)] = sg + j

    def fire_chunk(par, g):
        pltpu.async_copy(dtab_hbm.at[idxv[par].at[g]], dbufs[g], gsems[g])
        pltpu.async_copy(etab_hbm.at[idxv[par].at[g]], ebufs[g], gsems[g])

    def wait_chunk(g):
        pltpu.make_async_copy(
            dtab_hbm.at[pl.ds(0, _CP)], dbufs[g], gsems[g]).wait()
        pltpu.make_async_copy(
            etab_hbm.at[pl.ds(0, _CP)], ebufs[g], gsems[g]).wait()

    def row_phase(i, cur, nxt):
        have_next = i + 1 < _RPW

        @pl.when(have_next)
        def _():
            wait_x(nxt)
            compute_idx_w(nxt)

        @pl.when(i + 2 < _RPW)
        def _():
            load_x(i + 2, cur)

        zero = jnp.zeros((_L,), jnp.float32)
        acc = (zero,) * 10
        for g in range(_G):
            wait_chunk(g)

            def pair_body(q, carry, g=g):
                accs = list(carry)
                wq = wv[cur][pl.ds(_CP * g + _L * q, _L)]
                for k in range(_L):
                    p = _L * q + k
                    wbc = jnp.broadcast_to(wq[k], (_L,))
                    accs[0] = accs[0] + wbc * dbufs[g][p, pl.ds(0, _L)]
                    accs[1] = accs[1] + wbc * dbufs[g][p, pl.ds(_L, _L)]
                    for m in range(8):
                        accs[2 + m] = (accs[2 + m]
                                       + wbc * ebufs[g][p, pl.ds(_L * m, _L)])
                return tuple(accs)

            acc = lax.fori_loop(0, _CP // _L, pair_body, acc)

            @pl.when(have_next)
            def _(g=g):
                fire_chunk(nxt, g)

        for m in range(8):
            tv[pl.ds(_L * m, _L)] = jnp.exp(acc[2 + m])
        for h in range(2):
            o4 = (_L * h + lanes) * 4
            g0 = plsc.load_gather(tv, [o4])
            g1 = plsc.load_gather(tv, [o4 + 1])
            g2 = plsc.load_gather(tv, [o4 + 2])
            g3 = plsc.load_gather(tv, [o4 + 3])
            outv[pl.ds(32 * i + _L * h, _L)] = \
                (acc[h] + g0 - g2) + 0.25 * (g1 - g3)

    xv[0][pl.ds(96, _L)] = jnp.full((_L,), 0.5, jnp.float32)
    xv[1][pl.ds(96, _L)] = jnp.full((_L,), 0.5, jnp.float32)

    load_x(0, 0)
    wait_x(0)
    compute_idx_w(0)
    load_x(1, 1)
    for g in range(_G):
        fire_chunk(0, g)

    def two_rows(ii, _):
        row_phase(2 * ii, 0, 1)
        row_phase(2 * ii + 1, 1, 0)
        return 0

    lax.fori_loop(0, _RPW // 2, two_rows, 0)

    pltpu.sync_copy(outv, out_hbm.at[pl.ds(row0 * 32, _RPW * 32)])


_sc_kernel = functools.partial(
    pl.kernel,
    mesh=plsc.VectorSubcoreMesh(core_axis_name="c", subcore_axis_name="s"),
    out_type=jax.ShapeDtypeStruct((_B * 32,), jnp.float32),
    compiler_params=pltpu.CompilerParams(
        needs_layout_passes=False, use_tc_tiling_on_sc=False),
    scratch_types=[
        [pltpu.VMEM((112,), jnp.float32) for _ in range(2)],
        pltpu.VMEM((112,), jnp.float32),
        pltpu.VMEM((112,), jnp.int32),
        [pltpu.VMEM((_G, _CP), jnp.int32) for _ in range(2)],
        [pltpu.VMEM((_NPAIR,), jnp.float32) for _ in range(2)],
        [pltpu.VMEM((_CP, 32), jnp.float32) for _ in range(_G)],
        [pltpu.VMEM((_CP, 128), jnp.float32) for _ in range(_G)],
        pltpu.VMEM((128,), jnp.float32),
        pltpu.VMEM((_RPW * 32,), jnp.float32),
        pltpu.SemaphoreType.DMA,
        [pltpu.SemaphoreType.DMA for _ in range(_G)],
    ],
)(_sc_body)


def kernel(input_tensor, direct_table, exp_table):
    xp = jnp.pad(input_tensor, ((0, 0), (0, 4))).reshape(-1)
    out = _sc_kernel(xp, direct_table, exp_table)
    return out.reshape(_B, 32)


# speedup vs baseline: 1.9548x; 1.6823x over previous
import functools

import jax
import jax.numpy as jnp
from jax import lax
from jax.experimental import pallas as pl
from jax.experimental.pallas import tpu as pltpu
from jax.experimental.pallas import tpu_sc as plsc

_B = 1024
_D = 100
_DEN = 1003
_NPAIR = 4 * _D
_NW = 32
_RPW = _B // _NW
_L = 16
_G = 5
_CP = _NPAIR // _G


def _sc_body(x_hbm, dtab_hbm, etab_hbm, out_hbm,
             xv, uv, sv, idxv, wv, dbufs, ebufs, tv, outv,
             xsem, gsems):
    wid = lax.axis_index("s") * 2 + lax.axis_index("c")
    lanes = lax.iota(jnp.int32, _L)
    row0 = wid * _RPW

    def load_x(i, par):
        return pltpu.async_copy(
            x_hbm.at[pl.ds((row0 + i) * 104, 104)],
            xv[par].at[pl.ds(0, 104)], xsem)

    def wait_x(par):
        pltpu.make_async_copy(
            x_hbm.at[pl.ds(0, 104)], xv[par].at[pl.ds(0, 104)], xsem).wait()

    def compute_idx_w(par):
        for c in range(7):
            xx = xv[par][pl.ds(_L * c, _L)]
            t = xx * 1000.0
            bse = t.astype(jnp.int32)
            uv[pl.ds(_L * c, _L)] = t - bse.astype(jnp.float32)
            sv[pl.ds(_L * c, _L)] = (_L * c + lanes) * _DEN + bse
        for c in range(25):
            p = _L * c + lanes
            dq = lax.shift_right_logical(p, 2)
            j = lax.bitwise_and(p, 3)
            sg = plsc.load_gather(sv, [dq])
            ug = plsc.load_gather(uv, [dq])
            u2 = ug * ug
            u3 = u2 * ug
            om = 1.0 - ug
            w0 = om * om * om * (1.0 / 6.0)
            w1 = 0.5 * u3 - u2 + (2.0 / 3.0)
            w2 = -0.5 * u3 + 0.5 * u2 + 0.5 * ug + (1.0 / 6.0)
            w3 = u3 * (1.0 / 6.0)
            w = jnp.where(j == 0, w0,
                          jnp.where(j == 1, w1,
                                    jnp.where(j == 2, w2, w3)))
            wv[par][pl.ds(_L * c, _L)] = w
            idxv[par][(_L * c) // _CP, pl.ds((_L * c) % _CP, _L)] = sg + j

    def fire_chunk(par, g):
        pltpu.async_copy(dtab_hbm.at[idxv[par].at[g]], dbufs[g], gsems[g])
        pltpu.async_copy(etab_hbm.at[idxv[par].at[g]], ebufs[g], gsems[g])

    def wait_chunk(g):
        pltpu.make_async_copy(
            dtab_hbm.at[pl.ds(0, _CP)], dbufs[g], gsems[g]).wait()
        pltpu.make_async_copy(
            etab_hbm.at[pl.ds(0, _CP)], ebufs[g], gsems[g]).wait()

    def row_phase(i, cur, nxt):
        have_next = i + 1 < _RPW

        @pl.when(have_next)
        def _():
            wait_x(nxt)
            compute_idx_w(nxt)

        @pl.when(i + 2 < _RPW)
        def _():
            load_x(i + 2, cur)

        zero = jnp.zeros((_L,), jnp.float32)
        acc = (zero,) * 10
        for g in range(_G):
            wait_chunk(g)

            def pair_body(q, carry, g=g):
                accs = list(carry)
                for k in range(4):
                    p = 4 * q + k
                    wbc = plsc.load_gather(
                        wv[cur],
                        [jnp.full((_L,), _CP * g + 0, jnp.int32) + p])
                    accs[0] = accs[0] + wbc * dbufs[g][p, pl.ds(0, _L)]
                    accs[1] = accs[1] + wbc * dbufs[g][p, pl.ds(_L, _L)]
                    for m in range(8):
                        accs[2 + m] = (accs[2 + m]
                                       + wbc * ebufs[g][p, pl.ds(_L * m, _L)])
                return tuple(accs)

            acc = lax.fori_loop(0, _CP // 4, pair_body, acc)

            @pl.when(have_next)
            def _(g=g):
                fire_chunk(nxt, g)

        for m in range(8):
            tv[pl.ds(_L * m, _L)] = jnp.exp(acc[2 + m])
        for h in range(2):
            o4 = (_L * h + lanes) * 4
            g0 = plsc.load_gather(tv, [o4])
            g1 = plsc.load_gather(tv, [o4 + 1])
            g2 = plsc.load_gather(tv, [o4 + 2])
            g3 = plsc.load_gather(tv, [o4 + 3])
            outv[pl.ds(32 * i + _L * h, _L)] = \
                (acc[h] + g0 - g2) + 0.25 * (g1 - g3)

    xv[0][pl.ds(96, _L)] = jnp.full((_L,), 0.5, jnp.float32)
    xv[1][pl.ds(96, _L)] = jnp.full((_L,), 0.5, jnp.float32)

    load_x(0, 0)
    wait_x(0)
    compute_idx_w(0)
    load_x(1, 1)
    for g in range(_G):
        fire_chunk(0, g)

    def two_rows(ii, _):
        row_phase(2 * ii, 0, 1)
        row_phase(2 * ii + 1, 1, 0)
        return 0

    lax.fori_loop(0, _RPW // 2, two_rows, 0)

    pltpu.sync_copy(outv, out_hbm.at[pl.ds(row0 * 32, _RPW * 32)])


_sc_kernel = functools.partial(
    pl.kernel,
    mesh=plsc.VectorSubcoreMesh(core_axis_name="c", subcore_axis_name="s"),
    out_type=jax.ShapeDtypeStruct((_B * 32,), jnp.float32),
    compiler_params=pltpu.CompilerParams(
        needs_layout_passes=False, use_tc_tiling_on_sc=False),
    scratch_types=[
        [pltpu.VMEM((112,), jnp.float32) for _ in range(2)],
        pltpu.VMEM((112,), jnp.float32),
        pltpu.VMEM((112,), jnp.int32),
        [pltpu.VMEM((_G, _CP), jnp.int32) for _ in range(2)],
        [pltpu.VMEM((_NPAIR,), jnp.float32) for _ in range(2)],
        [pltpu.VMEM((_CP, 32), jnp.float32) for _ in range(_G)],
        [pltpu.VMEM((_CP, 128), jnp.float32) for _ in range(_G)],
        pltpu.VMEM((128,), jnp.float32),
        pltpu.VMEM((_RPW * 32,), jnp.float32),
        pltpu.SemaphoreType.DMA,
        [pltpu.SemaphoreType.DMA for _ in range(_G)],
    ],
)(_sc_body)


def kernel(input_tensor, direct_table, exp_table):
    xp = jnp.pad(input_tensor, ((0, 0), (0, 4))).reshape(-1)
    out = _sc_kernel(xp, direct_table, exp_table)
    return out.reshape(_B, 32)
